# baseline (speedup 1.0000x reference)
import jax
import jax.numpy as jnp
from jax.experimental import pallas as pl
from jax.experimental.pallas import tpu as pltpu
from jax.experimental.pallas import tpu_sc as plsc

EMBED_DIM = 64
RANK_DIM = 8
SCALING = 16.0 / 8.0
GATHER_WINDOW = 256
OUT_BATCH = 128


def _tc_m_stack(a_t, b_t):
    def body(at_ref, bt_ref, m_ref):
        eye = (jax.lax.broadcasted_iota(jnp.int32, (EMBED_DIM, EMBED_DIM), 0)
               == jax.lax.broadcasted_iota(
                   jnp.int32, (EMBED_DIM, EMBED_DIM), 1)).astype(jnp.float32)
        m = eye + SCALING * jnp.dot(at_ref[...], bt_ref[...],
                                    preferred_element_type=jnp.float32)
        m_ref[...] = jnp.concatenate([m, m], axis=0)

    return pl.pallas_call(
        body,
        out_shape=jax.ShapeDtypeStruct((2 * EMBED_DIM, EMBED_DIM),
                                       jnp.float32),
    )(a_t, b_t)


def _sc_gather(table_wide, idx_half):
    n = idx_half.shape[0]
    width = table_wide.shape[1]
    indices = idx_half.reshape(1, n)
    mesh = plsc.VectorSubcoreMesh(core_axis_name="core",
                                  subcore_axis_name="subcore")

    @pl.kernel(out_type=jax.ShapeDtypeStruct((n, width), table_wide.dtype),
               mesh=mesh)
    def gather_kernel(tab_hbm, i_hbm, o_hbm):
        def body(i_vmem, o_vmem):
            pltpu.sync_copy(tab_hbm.at[i_vmem.at[0]], o_vmem)

        pltpu.emit_pipeline(
            body,
            grid=(n // GATHER_WINDOW,),
            in_specs=[pl.BlockSpec((1, GATHER_WINDOW), lambda i: (0, i))],
            out_specs=[pl.BlockSpec((GATHER_WINDOW, width),
                                    lambda i: (i, 0))],
            core_axis_name=("core", "subcore"),
            dimension_semantics=(pltpu.PARALLEL,),
        )(i_hbm, o_hbm)

    return gather_kernel(table_wide, indices)


def _tc_select_lora(g_wide, par_t, m_stack, bsz, seq):
    rows_per_block = OUT_BATCH * seq
    par_cols = rows_per_block // 128

    def body(g_ref, p_ref, m_ref, o_ref):
        gb = g_ref[...]
        parts = []
        for j in range(par_cols):
            lo, hi = j * 128, (j + 1) * 128
            p = p_ref[0, :, j:j + 1]
            mask = jnp.concatenate(
                [jnp.broadcast_to(1.0 - p, (128, EMBED_DIM)),
                 jnp.broadcast_to(p, (128, EMBED_DIM))], axis=1)
            parts.append(gb[lo:hi] * mask)
        sel = jnp.concatenate(parts, axis=0)
        out = jnp.dot(sel, m_ref[...], preferred_element_type=jnp.float32)
        o_ref[...] = out.reshape(OUT_BATCH, seq, EMBED_DIM)

    return pl.pallas_call(
        body,
        grid=(bsz // OUT_BATCH,),
        in_specs=[
            pl.BlockSpec((rows_per_block, 2 * EMBED_DIM), lambda i: (i, 0)),
            pl.BlockSpec((1, 128, par_cols), lambda i: (i, 0, 0)),
            pl.BlockSpec((2 * EMBED_DIM, EMBED_DIM), lambda i: (0, 0)),
        ],
        out_specs=pl.BlockSpec((OUT_BATCH, seq, EMBED_DIM),
                               lambda i: (i, 0, 0)),
        out_shape=jax.ShapeDtypeStruct((bsz, seq, EMBED_DIM), jnp.float32),
    )(g_wide, par_t, m_stack)


def kernel(x, table, lora_A, lora_B):
    bsz, seq = x.shape
    n = bsz * seq
    par_cols = OUT_BATCH * seq // 128
    idx = x.reshape(-1).astype(jnp.int32)
    par_t = ((idx & 1).astype(jnp.float32)
             .reshape(n // 128, 128).T
             .reshape(128, bsz // OUT_BATCH, par_cols)
             .transpose(1, 0, 2))
    m_stack = _tc_m_stack(lora_A.T, lora_B.T)
    table_wide = table.reshape(table.shape[0] // 2, 2 * EMBED_DIM)
    g_wide = _sc_gather(table_wide, idx >> 1)
    return _tc_select_lora(g_wide, par_t, m_stack, bsz, seq)


# speedup vs baseline: 1.0116x; 1.0116x over previous
import jax
import jax.numpy as jnp
from jax.experimental import pallas as pl
from jax.experimental.pallas import tpu as pltpu
from jax.experimental.pallas import tpu_sc as plsc

EMBED_DIM = 64
RANK_DIM = 8
SCALING = 16.0 / 8.0
GATHER_WINDOW = 256
OUT_BATCH = 256


def _tc_m_stack(a_t, b_t):
    def body(at_ref, bt_ref, m_ref):
        eye = (jax.lax.broadcasted_iota(jnp.int32, (EMBED_DIM, EMBED_DIM), 0)
               == jax.lax.broadcasted_iota(
                   jnp.int32, (EMBED_DIM, EMBED_DIM), 1)).astype(jnp.float32)
        m = eye + SCALING * jnp.dot(at_ref[...], bt_ref[...],
                                    preferred_element_type=jnp.float32)
        m_ref[...] = jnp.concatenate([m, m], axis=0)

    return pl.pallas_call(
        body,
        out_shape=jax.ShapeDtypeStruct((2 * EMBED_DIM, EMBED_DIM),
                                       jnp.float32),
    )(a_t, b_t)


def _sc_gather(table_wide, idx_half):
    n = idx_half.shape[0]
    width = table_wide.shape[1]
    indices = idx_half.reshape(1, n)
    mesh = plsc.VectorSubcoreMesh(core_axis_name="core",
                                  subcore_axis_name="subcore")

    @pl.kernel(out_type=jax.ShapeDtypeStruct((n, width), table_wide.dtype),
               mesh=mesh)
    def gather_kernel(tab_hbm, i_hbm, o_hbm):
        def body(i_vmem, o_vmem):
            pltpu.sync_copy(tab_hbm.at[i_vmem.at[0]], o_vmem)

        pltpu.emit_pipeline(
            body,
            grid=(n // GATHER_WINDOW,),
            in_specs=[pl.BlockSpec((1, GATHER_WINDOW), lambda i: (0, i))],
            out_specs=[pl.BlockSpec((GATHER_WINDOW, width),
                                    lambda i: (i, 0))],
            core_axis_name=("core", "subcore"),
            dimension_semantics=(pltpu.PARALLEL,),
        )(i_hbm, o_hbm)

    return gather_kernel(table_wide, indices)


def _tc_select_lora(g_wide, par_t, m_stack, bsz, seq):
    rows_per_block = OUT_BATCH * seq
    par_cols = rows_per_block // 128

    def body(g_ref, p_ref, m_ref, o_ref):
        gb = g_ref[...]
        parts = []
        for j in range(par_cols):
            lo, hi = j * 128, (j + 1) * 128
            p = p_ref[0, :, j:j + 1]
            mask = jnp.concatenate(
                [jnp.broadcast_to(1.0 - p, (128, EMBED_DIM)),
                 jnp.broadcast_to(p, (128, EMBED_DIM))], axis=1)
            parts.append(gb[lo:hi] * mask)
        sel = jnp.concatenate(parts, axis=0)
        out = jnp.dot(sel, m_ref[...], preferred_element_type=jnp.float32)
        o_ref[...] = out.reshape(OUT_BATCH, seq, EMBED_DIM)

    return pl.pallas_call(
        body,
        grid=(bsz // OUT_BATCH,),
        in_specs=[
            pl.BlockSpec((rows_per_block, 2 * EMBED_DIM), lambda i: (i, 0)),
            pl.BlockSpec((1, 128, par_cols), lambda i: (i, 0, 0)),
            pl.BlockSpec((2 * EMBED_DIM, EMBED_DIM), lambda i: (0, 0)),
        ],
        out_specs=pl.BlockSpec((OUT_BATCH, seq, EMBED_DIM),
                               lambda i: (i, 0, 0)),
        out_shape=jax.ShapeDtypeStruct((bsz, seq, EMBED_DIM), jnp.float32),
    )(g_wide, par_t, m_stack)


def kernel(x, table, lora_A, lora_B):
    bsz, seq = x.shape
    n = bsz * seq
    par_cols = OUT_BATCH * seq // 128
    idx = x.reshape(-1).astype(jnp.int32)
    par_t = ((idx & 1).astype(jnp.float32)
             .reshape(n // 128, 128).T
             .reshape(128, bsz // OUT_BATCH, par_cols)
             .transpose(1, 0, 2))
    m_stack = _tc_m_stack(lora_A.T, lora_B.T)
    table_wide = table.reshape(table.shape[0] // 2, 2 * EMBED_DIM)
    g_wide = _sc_gather(table_wide, idx >> 1)
    return _tc_select_lora(g_wide, par_t, m_stack, bsz, seq)
